# initial kernel scaffold (unmeasured)
import jax
import jax.numpy as jnp
from jax import lax
from jax.experimental import pallas as pl
from jax.experimental.pallas import tpu as pltpu


def kernel(
    x,
):
    def body(*refs):
        pass

    out_shape = jax.ShapeDtypeStruct(..., jnp.float32)
    return pl.pallas_call(body, out_shape=out_shape)(...)


# baseline (device time: 222299 ns/iter reference)
import jax
import jax.numpy as jnp
from jax import lax
from jax.experimental import pallas as pl
from jax.experimental.pallas import tpu as pltpu

N_DEV = 8
M = 2048
N_PER = 512


def kernel(x):
    x3 = jnp.moveaxis(x[0].astype(jnp.bfloat16).reshape(M, N_DEV, N_PER), 1, 0)

    def body(x_ref, out_ref, stage_ref, recv_ref, send_sem, recv_sems):
        my = lax.axis_index("i")
        left = lax.rem(my + N_DEV - 1, N_DEV)
        right = lax.rem(my + 1, N_DEV)

        barrier_sem = pltpu.get_barrier_semaphore()
        for nbr in (left, right):
            pl.semaphore_signal(
                barrier_sem,
                inc=1,
                device_id=(nbr,),
                device_id_type=pl.DeviceIdType.MESH,
            )
        pl.semaphore_wait(barrier_sem, 2)

        for s in range(N_DEV - 1):
            c = lax.rem(my + N_DEV - 1 - s, N_DEV)
            if s == 0:
                stage_ref[...] = x_ref[c]
            else:
                stage_ref[...] = recv_ref[s - 1] + x_ref[c]
            rdma = pltpu.make_async_remote_copy(
                src_ref=stage_ref,
                dst_ref=recv_ref.at[s],
                send_sem=send_sem,
                recv_sem=recv_sems.at[s],
                device_id=(right,),
                device_id_type=pl.DeviceIdType.MESH,
            )
            rdma.start()
            rdma.wait()

        out_ref[...] = recv_ref[N_DEV - 2].astype(jnp.float32) + x_ref[
            my
        ].astype(jnp.float32)

    return pl.pallas_call(
        body,
        out_shape=jax.ShapeDtypeStruct((M, N_PER), jnp.float32),
        in_specs=[pl.BlockSpec(memory_space=pltpu.VMEM)],
        out_specs=pl.BlockSpec(memory_space=pltpu.VMEM),
        scratch_shapes=[
            pltpu.VMEM((M, N_PER), jnp.bfloat16),
            pltpu.VMEM((N_DEV - 1, M, N_PER), jnp.bfloat16),
            pltpu.SemaphoreType.DMA,
            pltpu.SemaphoreType.DMA((N_DEV - 1,)),
        ],
        compiler_params=pltpu.CompilerParams(collective_id=0),
    )(x3)


# device time: 133227 ns/iter; 1.6686x vs baseline; 1.6686x over previous
import jax
import jax.numpy as jnp
from jax import lax
from jax.experimental import pallas as pl
from jax.experimental.pallas import tpu as pltpu

N_DEV = 8
M = 2048
N_PER = 512
H = M // 2


def kernel(x):
    x4 = jnp.moveaxis(x[0].astype(jnp.bfloat16).reshape(M, N_DEV, N_PER), 1, 0)
    x4 = x4.reshape(N_DEV, 2, H, N_PER)

    def body(
        x_ref,
        out_ref,
        stage_cw,
        stage_ccw,
        recv_cw,
        recv_ccw,
        send_sems_cw,
        send_sems_ccw,
        recv_sems_cw,
        recv_sems_ccw,
    ):
        my = lax.axis_index("i")
        left = lax.rem(my + N_DEV - 1, N_DEV)
        right = lax.rem(my + 1, N_DEV)

        barrier_sem = pltpu.get_barrier_semaphore()
        for nbr in (left, right):
            pl.semaphore_signal(
                barrier_sem,
                inc=1,
                device_id=(nbr,),
                device_id_type=pl.DeviceIdType.MESH,
            )
        pl.semaphore_wait(barrier_sem, 2)

        cw = []
        ccw = []
        for s in range(N_DEV - 1):
            slot = s % 2
            if s >= 2:
                cw[s - 2].wait_send()
                ccw[s - 2].wait_send()
            c_cw = lax.rem(my + N_DEV - 1 - s, N_DEV)
            c_ccw = lax.rem(my + 1 + s, N_DEV)

            if s == 0:
                stage_cw[slot] = x_ref[c_cw, 0]
            else:
                cw[s - 1].wait_recv()
                stage_cw[slot] = recv_cw[s - 1] + x_ref[c_cw, 0]
            r = pltpu.make_async_remote_copy(
                src_ref=stage_cw.at[slot],
                dst_ref=recv_cw.at[s],
                send_sem=send_sems_cw.at[slot],
                recv_sem=recv_sems_cw.at[s],
                device_id=(right,),
                device_id_type=pl.DeviceIdType.MESH,
            )
            r.start()
            cw.append(r)

            if s == 0:
                stage_ccw[slot] = x_ref[c_ccw, 1]
            else:
                ccw[s - 1].wait_recv()
                stage_ccw[slot] = recv_ccw[s - 1] + x_ref[c_ccw, 1]
            r = pltpu.make_async_remote_copy(
                src_ref=stage_ccw.at[slot],
                dst_ref=recv_ccw.at[s],
                send_sem=send_sems_ccw.at[slot],
                recv_sem=recv_sems_ccw.at[s],
                device_id=(left,),
                device_id_type=pl.DeviceIdType.MESH,
            )
            r.start()
            ccw.append(r)

        last = N_DEV - 2
        cw[last].wait_recv()
        out_ref[0:H, :] = recv_cw[last].astype(jnp.float32) + x_ref[my, 0].astype(
            jnp.float32
        )
        ccw[last].wait_recv()
        out_ref[H:M, :] = recv_ccw[last].astype(jnp.float32) + x_ref[
            my, 1
        ].astype(jnp.float32)

        for s in (last - 1, last):
            cw[s].wait_send()
            ccw[s].wait_send()

    return pl.pallas_call(
        body,
        out_shape=jax.ShapeDtypeStruct((M, N_PER), jnp.float32),
        in_specs=[pl.BlockSpec(memory_space=pltpu.VMEM)],
        out_specs=pl.BlockSpec(memory_space=pltpu.VMEM),
        scratch_shapes=[
            pltpu.VMEM((2, H, N_PER), jnp.bfloat16),
            pltpu.VMEM((2, H, N_PER), jnp.bfloat16),
            pltpu.VMEM((N_DEV - 1, H, N_PER), jnp.bfloat16),
            pltpu.VMEM((N_DEV - 1, H, N_PER), jnp.bfloat16),
            pltpu.SemaphoreType.DMA((2,)),
            pltpu.SemaphoreType.DMA((2,)),
            pltpu.SemaphoreType.DMA((N_DEV - 1,)),
            pltpu.SemaphoreType.DMA((N_DEV - 1,)),
        ],
        compiler_params=pltpu.CompilerParams(collective_id=0),
    )(x4)


# device time: 99498 ns/iter; 2.2342x vs baseline; 1.3390x over previous
import jax
import jax.numpy as jnp
from jax import lax
from jax.experimental import pallas as pl
from jax.experimental.pallas import tpu as pltpu

N_DEV = 8
M = 2048
N_PER = 512
H = M // 2


def kernel(x):
    def body(
        x_hbm,
        out_ref,
        xf_cw,
        xf_ccw,
        stage_cw,
        stage_ccw,
        recv_cw,
        recv_ccw,
        dma_sems_cw,
        dma_sems_ccw,
        send_sems_cw,
        send_sems_ccw,
        recv_sems_cw,
        recv_sems_ccw,
    ):
        my = lax.axis_index("i")
        left = lax.rem(my + N_DEV - 1, N_DEV)
        right = lax.rem(my + 1, N_DEV)

        def issue_fetch(n):
            slot = n % 2
            if n < N_DEV - 1:
                c_cw = lax.rem(my + N_DEV - 1 - n, N_DEV)
                c_ccw = lax.rem(my + 1 + n, N_DEV)
            else:
                c_cw = my
                c_ccw = my
            f_cw = pltpu.make_async_copy(
                x_hbm.at[0, 0:H, pl.ds(c_cw * N_PER, N_PER)],
                xf_cw.at[slot],
                dma_sems_cw.at[slot],
            )
            f_cw.start()
            f_ccw = pltpu.make_async_copy(
                x_hbm.at[0, H:M, pl.ds(c_ccw * N_PER, N_PER)],
                xf_ccw.at[slot],
                dma_sems_ccw.at[slot],
            )
            f_ccw.start()
            return f_cw, f_ccw

        fetches = [issue_fetch(0), issue_fetch(1)]

        barrier_sem = pltpu.get_barrier_semaphore()
        for nbr in (left, right):
            pl.semaphore_signal(
                barrier_sem,
                inc=1,
                device_id=(nbr,),
                device_id_type=pl.DeviceIdType.MESH,
            )
        pl.semaphore_wait(barrier_sem, 2)

        cw = []
        ccw = []
        for s in range(N_DEV - 1):
            slot = s % 2
            if s >= 2:
                cw[s - 2].wait_send()
                ccw[s - 2].wait_send()

            fetches[s][0].wait()
            if s == 0:
                stage_cw[slot] = xf_cw[slot].astype(jnp.bfloat16)
            else:
                cw[s - 1].wait_recv()
                stage_cw[slot] = recv_cw[s - 1] + xf_cw[slot].astype(
                    jnp.bfloat16
                )
            r = pltpu.make_async_remote_copy(
                src_ref=stage_cw.at[slot],
                dst_ref=recv_cw.at[s],
                send_sem=send_sems_cw.at[slot],
                recv_sem=recv_sems_cw.at[s],
                device_id=(right,),
                device_id_type=pl.DeviceIdType.MESH,
            )
            r.start()
            cw.append(r)

            fetches[s][1].wait()
            if s == 0:
                stage_ccw[slot] = xf_ccw[slot].astype(jnp.bfloat16)
            else:
                ccw[s - 1].wait_recv()
                stage_ccw[slot] = recv_ccw[s - 1] + xf_ccw[slot].astype(
                    jnp.bfloat16
                )
            r = pltpu.make_async_remote_copy(
                src_ref=stage_ccw.at[slot],
                dst_ref=recv_ccw.at[s],
                send_sem=send_sems_ccw.at[slot],
                recv_sem=recv_sems_ccw.at[s],
                device_id=(left,),
                device_id_type=pl.DeviceIdType.MESH,
            )
            r.start()
            ccw.append(r)

            if s + 2 <= N_DEV - 1:
                fetches.append(issue_fetch(s + 2))

        last = N_DEV - 2
        my_slot = (N_DEV - 1) % 2
        fetches[N_DEV - 1][0].wait()
        cw[last].wait_recv()
        out_ref[0:H, :] = recv_cw[last].astype(jnp.float32) + xf_cw[my_slot]
        fetches[N_DEV - 1][1].wait()
        ccw[last].wait_recv()
        out_ref[H:M, :] = recv_ccw[last].astype(jnp.float32) + xf_ccw[my_slot]

        for s in (last - 1, last):
            cw[s].wait_send()
            ccw[s].wait_send()

    return pl.pallas_call(
        body,
        out_shape=jax.ShapeDtypeStruct((M, N_PER), jnp.float32),
        in_specs=[pl.BlockSpec(memory_space=pl.ANY)],
        out_specs=pl.BlockSpec(memory_space=pltpu.VMEM),
        scratch_shapes=[
            pltpu.VMEM((2, H, N_PER), jnp.float32),
            pltpu.VMEM((2, H, N_PER), jnp.float32),
            pltpu.VMEM((2, H, N_PER), jnp.bfloat16),
            pltpu.VMEM((2, H, N_PER), jnp.bfloat16),
            pltpu.VMEM((N_DEV - 1, H, N_PER), jnp.bfloat16),
            pltpu.VMEM((N_DEV - 1, H, N_PER), jnp.bfloat16),
            pltpu.SemaphoreType.DMA((2,)),
            pltpu.SemaphoreType.DMA((2,)),
            pltpu.SemaphoreType.DMA((2,)),
            pltpu.SemaphoreType.DMA((2,)),
            pltpu.SemaphoreType.DMA((N_DEV - 1,)),
            pltpu.SemaphoreType.DMA((N_DEV - 1,)),
        ],
        compiler_params=pltpu.CompilerParams(collective_id=0),
    )(x)


# device time: 98760 ns/iter; 2.2509x vs baseline; 1.0075x over previous
import jax
import jax.numpy as jnp
from jax import lax
from jax.experimental import pallas as pl
from jax.experimental.pallas import tpu as pltpu

N_DEV = 8
M = 2048
N_PER = 512
H = M // 2


def kernel(x):
    def body(
        x_hbm,
        out_ref,
        xf_cw,
        xf_ccw,
        stage_cw,
        stage_ccw,
        recv_cw,
        recv_ccw,
        dma_sems_cw,
        dma_sems_ccw,
        send_sems_cw,
        send_sems_ccw,
        recv_sems_cw,
        recv_sems_ccw,
    ):
        my = lax.axis_index("i")

        def ring2id(p):
            p = lax.rem(p + N_DEV, N_DEV)
            return jnp.where(p < 4, p, 11 - p)

        pos = jnp.where(my < 4, my, 11 - my)
        left = ring2id(pos - 1)
        right = ring2id(pos + 1)

        def issue_fetch(n):
            slot = n % 2
            if n < N_DEV - 1:
                c_cw = ring2id(pos - 1 - n)
                c_ccw = ring2id(pos + 1 + n)
            else:
                c_cw = my
                c_ccw = my
            f_cw = pltpu.make_async_copy(
                x_hbm.at[0, 0:H, pl.ds(c_cw * N_PER, N_PER)],
                xf_cw.at[slot],
                dma_sems_cw.at[slot],
            )
            f_cw.start()
            f_ccw = pltpu.make_async_copy(
                x_hbm.at[0, H:M, pl.ds(c_ccw * N_PER, N_PER)],
                xf_ccw.at[slot],
                dma_sems_ccw.at[slot],
            )
            f_ccw.start()
            return f_cw, f_ccw

        fetches = [issue_fetch(0), issue_fetch(1)]

        barrier_sem = pltpu.get_barrier_semaphore()
        for nbr in (left, right):
            pl.semaphore_signal(
                barrier_sem,
                inc=1,
                device_id=(nbr,),
                device_id_type=pl.DeviceIdType.MESH,
            )
        pl.semaphore_wait(barrier_sem, 2)

        cw = []
        ccw = []
        for s in range(N_DEV - 1):
            slot = s % 2
            if s >= 2:
                cw[s - 2].wait_send()
                ccw[s - 2].wait_send()

            fetches[s][0].wait()
            if s == 0:
                stage_cw[slot] = xf_cw[slot].astype(jnp.bfloat16)
            else:
                cw[s - 1].wait_recv()
                stage_cw[slot] = recv_cw[s - 1] + xf_cw[slot].astype(
                    jnp.bfloat16
                )
            r = pltpu.make_async_remote_copy(
                src_ref=stage_cw.at[slot],
                dst_ref=recv_cw.at[s],
                send_sem=send_sems_cw.at[slot],
                recv_sem=recv_sems_cw.at[s],
                device_id=(right,),
                device_id_type=pl.DeviceIdType.MESH,
            )
            r.start()
            cw.append(r)

            fetches[s][1].wait()
            if s == 0:
                stage_ccw[slot] = xf_ccw[slot].astype(jnp.bfloat16)
            else:
                ccw[s - 1].wait_recv()
                stage_ccw[slot] = recv_ccw[s - 1] + xf_ccw[slot].astype(
                    jnp.bfloat16
                )
            r = pltpu.make_async_remote_copy(
                src_ref=stage_ccw.at[slot],
                dst_ref=recv_ccw.at[s],
                send_sem=send_sems_ccw.at[slot],
                recv_sem=recv_sems_ccw.at[s],
                device_id=(left,),
                device_id_type=pl.DeviceIdType.MESH,
            )
            r.start()
            ccw.append(r)

            if s + 2 <= N_DEV - 1:
                fetches.append(issue_fetch(s + 2))

        last = N_DEV - 2
        my_slot = (N_DEV - 1) % 2
        fetches[N_DEV - 1][0].wait()
        cw[last].wait_recv()
        out_ref[0:H, :] = recv_cw[last].astype(jnp.float32) + xf_cw[my_slot]
        fetches[N_DEV - 1][1].wait()
        ccw[last].wait_recv()
        out_ref[H:M, :] = recv_ccw[last].astype(jnp.float32) + xf_ccw[my_slot]

        for s in (last - 1, last):
            cw[s].wait_send()
            ccw[s].wait_send()

    return pl.pallas_call(
        body,
        out_shape=jax.ShapeDtypeStruct((M, N_PER), jnp.float32),
        in_specs=[pl.BlockSpec(memory_space=pl.ANY)],
        out_specs=pl.BlockSpec(memory_space=pltpu.VMEM),
        scratch_shapes=[
            pltpu.VMEM((2, H, N_PER), jnp.float32),
            pltpu.VMEM((2, H, N_PER), jnp.float32),
            pltpu.VMEM((2, H, N_PER), jnp.bfloat16),
            pltpu.VMEM((2, H, N_PER), jnp.bfloat16),
            pltpu.VMEM((N_DEV - 1, H, N_PER), jnp.bfloat16),
            pltpu.VMEM((N_DEV - 1, H, N_PER), jnp.bfloat16),
            pltpu.SemaphoreType.DMA((2,)),
            pltpu.SemaphoreType.DMA((2,)),
            pltpu.SemaphoreType.DMA((2,)),
            pltpu.SemaphoreType.DMA((2,)),
            pltpu.SemaphoreType.DMA((N_DEV - 1,)),
            pltpu.SemaphoreType.DMA((N_DEV - 1,)),
        ],
        compiler_params=pltpu.CompilerParams(collective_id=0),
    )(x)


# device time: 87537 ns/iter; 2.5395x vs baseline; 1.1282x over previous
import jax
import jax.numpy as jnp
from jax import lax
from jax.experimental import pallas as pl
from jax.experimental.pallas import tpu as pltpu

N_DEV = 8
M = 2048
N_PER = 512
H = M // 2
NSUB = 2
HS = H // NSUB


def kernel(x):
    def body(
        x_hbm,
        out_ref,
        xf_cw,
        xf_ccw,
        stage_cw,
        stage_ccw,
        recv_cw,
        recv_ccw,
        dma_sems_cw,
        dma_sems_ccw,
        send_sems_cw,
        send_sems_ccw,
        recv_sems_cw,
        recv_sems_ccw,
    ):
        my = lax.axis_index("i")

        def ring2id(p):
            p = lax.rem(p + N_DEV, N_DEV)
            return jnp.where(p < 4, p, 11 - p)

        pos = jnp.where(my < 4, my, 11 - my)
        left = ring2id(pos - 1)
        right = ring2id(pos + 1)

        def issue_fetch(n):
            slot = n % 2
            if n < N_DEV - 1:
                c_cw = ring2id(pos - 1 - n)
                c_ccw = ring2id(pos + 1 + n)
            else:
                c_cw = my
                c_ccw = my
            f_cw = pltpu.make_async_copy(
                x_hbm.at[0, 0:H, pl.ds(c_cw * N_PER, N_PER)],
                xf_cw.at[slot],
                dma_sems_cw.at[slot],
            )
            f_cw.start()
            f_ccw = pltpu.make_async_copy(
                x_hbm.at[0, H:M, pl.ds(c_ccw * N_PER, N_PER)],
                xf_ccw.at[slot],
                dma_sems_ccw.at[slot],
            )
            f_ccw.start()
            return f_cw, f_ccw

        fetches = [issue_fetch(0), issue_fetch(1)]

        barrier_sem = pltpu.get_barrier_semaphore()
        for nbr in (left, right):
            pl.semaphore_signal(
                barrier_sem,
                inc=1,
                device_id=(nbr,),
                device_id_type=pl.DeviceIdType.MESH,
            )
        pl.semaphore_wait(barrier_sem, 2)

        cw = []
        ccw = []

        def sub(j):
            return pl.ds(j * HS, HS)

        def hop_sub(s, j, stage, recv, xf, rs, send_sems, recv_sems, tgt):
            slot = s % 2
            if s == 0:
                stage[slot, sub(j)] = xf[slot, sub(j)].astype(jnp.bfloat16)
            else:
                rs[s - 1][j].wait_recv()
                stage[slot, sub(j)] = recv[s - 1, sub(j)] + xf[
                    slot, sub(j)
                ].astype(jnp.bfloat16)
            r = pltpu.make_async_remote_copy(
                src_ref=stage.at[slot, sub(j)],
                dst_ref=recv.at[s, sub(j)],
                send_sem=send_sems.at[slot, j],
                recv_sem=recv_sems.at[s, j],
                device_id=(tgt,),
                device_id_type=pl.DeviceIdType.MESH,
            )
            r.start()
            return r

        for s in range(N_DEV - 1):
            if s >= 2:
                for j in range(NSUB):
                    cw[s - 2][j].wait_send()
                    ccw[s - 2][j].wait_send()
            fetches[s][0].wait()
            fetches[s][1].wait()
            cw_subs = []
            ccw_subs = []
            for j in range(NSUB):
                cw_subs.append(
                    hop_sub(
                        s, j, stage_cw, recv_cw, xf_cw, cw,
                        send_sems_cw, recv_sems_cw, right,
                    )
                )
                ccw_subs.append(
                    hop_sub(
                        s, j, stage_ccw, recv_ccw, xf_ccw, ccw,
                        send_sems_ccw, recv_sems_ccw, left,
                    )
                )
            cw.append(cw_subs)
            ccw.append(ccw_subs)

            if s + 2 <= N_DEV - 1:
                fetches.append(issue_fetch(s + 2))

        last = N_DEV - 2
        my_slot = (N_DEV - 1) % 2
        fetches[N_DEV - 1][0].wait()
        fetches[N_DEV - 1][1].wait()
        for j in range(NSUB):
            cw[last][j].wait_recv()
            out_ref[sub(j), :] = recv_cw[last, sub(j)].astype(
                jnp.float32
            ) + xf_cw[my_slot, sub(j)]
            ccw[last][j].wait_recv()
            out_ref[pl.ds(H + j * HS, HS), :] = recv_ccw[last, sub(j)].astype(
                jnp.float32
            ) + xf_ccw[my_slot, sub(j)]

        for s in (last - 1, last):
            for j in range(NSUB):
                cw[s][j].wait_send()
                ccw[s][j].wait_send()

    return pl.pallas_call(
        body,
        out_shape=jax.ShapeDtypeStruct((M, N_PER), jnp.float32),
        in_specs=[pl.BlockSpec(memory_space=pl.ANY)],
        out_specs=pl.BlockSpec(memory_space=pltpu.VMEM),
        scratch_shapes=[
            pltpu.VMEM((2, H, N_PER), jnp.float32),
            pltpu.VMEM((2, H, N_PER), jnp.float32),
            pltpu.VMEM((2, H, N_PER), jnp.bfloat16),
            pltpu.VMEM((2, H, N_PER), jnp.bfloat16),
            pltpu.VMEM((N_DEV - 1, H, N_PER), jnp.bfloat16),
            pltpu.VMEM((N_DEV - 1, H, N_PER), jnp.bfloat16),
            pltpu.SemaphoreType.DMA((2,)),
            pltpu.SemaphoreType.DMA((2,)),
            pltpu.SemaphoreType.DMA((2, NSUB)),
            pltpu.SemaphoreType.DMA((2, NSUB)),
            pltpu.SemaphoreType.DMA((N_DEV - 1, NSUB)),
            pltpu.SemaphoreType.DMA((N_DEV - 1, NSUB)),
        ],
        compiler_params=pltpu.CompilerParams(collective_id=0),
    )(x)
